# baseline (speedup 1.0000x reference)
import jax
import jax.numpy as jnp
from jax.experimental import pallas as pl
from jax.experimental.pallas import tpu as pltpu

N = 4096
F = 128
BI = 512
BJ = 512


def _make_hop_body(gi, gj, bi, bj, need_r, need_next, has_acc, relu):
    def body(*refs):
        it = iter(refs)
        A = next(it)
        x0 = next(it)
        x1 = next(it)
        accl = next(it) if has_acc else None
        accr = next(it) if (has_acc and need_r) else None
        W1 = next(it)
        b1 = next(it)
        W2 = next(it)
        b2 = next(it)
        outl = next(it)
        outr = next(it) if need_r else None
        nl = next(it) if need_next else None
        nr = next(it) if need_next else None
        yl = next(it)
        yr = next(it) if need_r else None

        i = pl.program_id(0)
        j = pl.program_id(1)
        a = A[...]

        x0b = x0[pl.ds(j * bj, bj), :].astype(jnp.bfloat16)
        part_l = jax.lax.dot_general(
            a, x0b, (((1,), (0,)), ((), ())), preferred_element_type=jnp.float32
        )

        @pl.when(j == 0)
        def _():
            yl[pl.ds(i * bi, bi), :] = part_l

        @pl.when(j != 0)
        def _():
            yl[pl.ds(i * bi, bi), :] += part_l

        if need_r:
            x1b = x1[pl.ds(i * bi, bi), :].astype(jnp.bfloat16)
            part_r = jax.lax.dot_general(
                a, x1b, (((0,), (0,)), ((), ())), preferred_element_type=jnp.float32
            )

            @pl.when(i == 0)
            def _():
                yr[pl.ds(j * bj, bj), :] = part_r

            @pl.when(i != 0)
            def _():
                yr[pl.ds(j * bj, bj), :] += part_r

        @pl.when((i == gi - 1) & (j == gj - 1))
        def _():
            W1v = W1[...].astype(jnp.bfloat16)
            W2v = W2[...].astype(jnp.bfloat16)
            bias = b1[...] + b2[...]
            ylv = yl[...]
            lm = ylv * x1[...]
            ol = (
                jnp.dot(ylv.astype(jnp.bfloat16), W1v, preferred_element_type=jnp.float32)
                + jnp.dot(lm.astype(jnp.bfloat16), W2v, preferred_element_type=jnp.float32)
                + bias
            )
            if has_acc:
                ol = ol + accl[...]
            if relu:
                ol = jnp.maximum(ol, 0.0)
            outl[...] = ol
            if need_next:
                nl[...] = ylv + lm
            if need_r:
                yrv = yr[...]
                rm = yrv * x0[...]
                orv = (
                    jnp.dot(yrv.astype(jnp.bfloat16), W1v, preferred_element_type=jnp.float32)
                    + jnp.dot(rm.astype(jnp.bfloat16), W2v, preferred_element_type=jnp.float32)
                    + bias
                )
                if has_acc:
                    orv = orv + accr[...]
                if relu:
                    orv = jnp.maximum(orv, 0.0)
                outr[...] = orv
                if need_next:
                    nr[...] = yrv + rm

    return body


def _hop(A, x0, x1, accs, W1, b1, W2, b2, *, need_r, need_next, relu):
    has_acc = accs is not None
    gi = N // BI
    gj = N // BJ
    full = pl.BlockSpec((N, F), lambda i, j: (0, 0))
    wspec = pl.BlockSpec((F, F), lambda i, j: (0, 0))
    bspec = pl.BlockSpec((1, F), lambda i, j: (0, 0))
    in_specs = [pl.BlockSpec((BI, BJ), lambda i, j: (i, j)), full, full]
    ops = [A, x0, x1]
    if has_acc:
        in_specs.append(full)
        ops.append(accs[0])
        if need_r:
            in_specs.append(full)
            ops.append(accs[1])
    in_specs += [wspec, bspec, wspec, bspec]
    ops += [W1, b1, W2, b2]

    n_outs = 1 + (1 if need_r else 0) + (2 if need_next else 0)
    out_shape = tuple(jax.ShapeDtypeStruct((N, F), jnp.float32) for _ in range(n_outs))
    out_specs = tuple(full for _ in range(n_outs))
    scratch = [pltpu.VMEM((N, F), jnp.float32)]
    if need_r:
        scratch.append(pltpu.VMEM((N, F), jnp.float32))

    return pl.pallas_call(
        _make_hop_body(gi, gj, BI, BJ, need_r, need_next, has_acc, relu),
        grid=(gi, gj),
        in_specs=in_specs,
        out_specs=out_specs,
        out_shape=out_shape,
        scratch_shapes=scratch,
    )(*ops)


def kernel(l_feat, r_feat, network, W1a, b1a, W2a, b2a, W1b, b1b, W2b, b2b):
    network = network.astype(jnp.bfloat16)
    b1a = b1a.reshape(1, F)
    b2a = b2a.reshape(1, F)
    b1b = b1b.reshape(1, F)
    b2b = b2b.reshape(1, F)

    ol, orv, nl, nr = _hop(
        network, r_feat, l_feat, None, W1a, b1a, W2a, b2a,
        need_r=True, need_next=True, relu=False,
    )
    y1, z1 = _hop(
        network, nr, nl, (ol, orv), W1a, b1a, W2a, b2a,
        need_r=True, need_next=False, relu=True,
    )
    ol2, or2, nl2, nr2 = _hop(
        network, z1, y1, None, W1b, b1b, W2b, b2b,
        need_r=True, need_next=True, relu=False,
    )
    (y2,) = _hop(
        network, nr2, nl2, (ol2,), W1b, b1b, W2b, b2b,
        need_r=False, need_next=False, relu=True,
    )
    return y2


# speedup vs baseline: 1.3629x; 1.3629x over previous
import jax
import jax.numpy as jnp
from jax.experimental import pallas as pl
from jax.experimental.pallas import tpu as pltpu

N = 4096
F = 128
BI = 1024
BJ = 1024


def _make_hop_body(gi, gj, bi, bj, need_r, need_next, has_acc, relu, bf_out):
    def body(*refs):
        it = iter(refs)
        A = next(it)
        x0s = next(it)
        x1s = next(it) if need_r else None
        x1f = next(it)
        x0f = next(it) if need_r else None
        accl = next(it) if has_acc else None
        accr = next(it) if (has_acc and need_r) else None
        W1 = next(it)
        b1 = next(it)
        W2 = next(it)
        b2 = next(it)
        outl = next(it)
        outr = next(it) if need_r else None
        nl = next(it) if need_next else None
        nr = next(it) if need_next else None
        nlbf = next(it) if (need_next or bf_out) else None
        nrbf = next(it) if (need_next or bf_out) else None
        yl = next(it)
        yr = next(it) if need_r else None

        i = pl.program_id(0)
        j = pl.program_id(1)
        a = A[...]

        part_l = jax.lax.dot_general(
            a, x0s[...], (((1,), (0,)), ((), ())),
            preferred_element_type=jnp.float32,
        )

        @pl.when(j == 0)
        def _():
            yl[pl.ds(i * bi, bi), :] = part_l

        @pl.when(j != 0)
        def _():
            yl[pl.ds(i * bi, bi), :] += part_l

        if need_r:
            part_r = jax.lax.dot_general(
                a, x1s[...], (((0,), (0,)), ((), ())),
                preferred_element_type=jnp.float32,
            )

            @pl.when(i == 0)
            def _():
                yr[pl.ds(j * bj, bj), :] = part_r

            @pl.when(i != 0)
            def _():
                yr[pl.ds(j * bj, bj), :] += part_r

        @pl.when(j == gj - 1)
        def _():
            W1v = W1[...].astype(jnp.bfloat16)
            W2v = W2[...].astype(jnp.bfloat16)
            bias = b1[...] + b2[...]
            ylv = yl[pl.ds(i * bi, bi), :]
            lm = ylv * x1f[pl.ds(i * bi, bi), :]
            ol = (
                jnp.dot(ylv.astype(jnp.bfloat16), W1v, preferred_element_type=jnp.float32)
                + jnp.dot(lm.astype(jnp.bfloat16), W2v, preferred_element_type=jnp.float32)
                + bias
            )
            if has_acc:
                ol = ol + accl[pl.ds(i * bi, bi), :]
            if relu:
                ol = jnp.maximum(ol, 0.0)
            outl[pl.ds(i * bi, bi), :] = ol
            if need_next:
                nx = ylv + lm
                nl[pl.ds(i * bi, bi), :] = nx
                nlbf[pl.ds(i * bi, bi), :] = nx.astype(jnp.bfloat16)
            if bf_out:
                nlbf[pl.ds(i * bi, bi), :] = ol.astype(jnp.bfloat16)

        if need_r:
            @pl.when(i == gi - 1)
            def _():
                W1v = W1[...].astype(jnp.bfloat16)
                W2v = W2[...].astype(jnp.bfloat16)
                bias = b1[...] + b2[...]
                yrv = yr[pl.ds(j * bj, bj), :]
                rm = yrv * x0f[pl.ds(j * bj, bj), :]
                orv = (
                    jnp.dot(yrv.astype(jnp.bfloat16), W1v, preferred_element_type=jnp.float32)
                    + jnp.dot(rm.astype(jnp.bfloat16), W2v, preferred_element_type=jnp.float32)
                    + bias
                )
                if has_acc:
                    orv = orv + accr[pl.ds(j * bj, bj), :]
                if relu:
                    orv = jnp.maximum(orv, 0.0)
                outr[pl.ds(j * bj, bj), :] = orv
                if need_next:
                    nx = yrv + rm
                    nr[pl.ds(j * bj, bj), :] = nx
                    nrbf[pl.ds(j * bj, bj), :] = nx.astype(jnp.bfloat16)
                if bf_out:
                    nrbf[pl.ds(j * bj, bj), :] = orv.astype(jnp.bfloat16)

    return body


def _hop(A16, x0bf, x1bf, x1f, x0f, accs, W1, b1, W2, b2,
         *, need_r, need_next, relu, bf_out):
    has_acc = accs is not None
    gi = N // BI
    gj = N // BJ
    full = pl.BlockSpec((N, F), lambda i, j: (0, 0))
    wspec = pl.BlockSpec((F, F), lambda i, j: (0, 0))
    bspec = pl.BlockSpec((1, F), lambda i, j: (0, 0))
    in_specs = [
        pl.BlockSpec((BI, BJ), lambda i, j: (i, j)),
        pl.BlockSpec((BJ, F), lambda i, j: (j, 0)),
    ]
    ops = [A16, x0bf]
    if need_r:
        in_specs.append(pl.BlockSpec((BI, F), lambda i, j: (i, 0)))
        ops.append(x1bf)
    in_specs.append(full)
    ops.append(x1f)
    if need_r:
        in_specs.append(full)
        ops.append(x0f)
    if has_acc:
        in_specs.append(full)
        ops.append(accs[0])
        if need_r:
            in_specs.append(full)
            ops.append(accs[1])
    in_specs += [wspec, bspec, wspec, bspec]
    ops += [W1, b1, W2, b2]

    n_f32 = 1 + (1 if need_r else 0) + (2 if need_next else 0)
    n_bf = 2 if (need_next or bf_out) else 0
    out_shape = tuple(
        [jax.ShapeDtypeStruct((N, F), jnp.float32) for _ in range(n_f32)]
        + [jax.ShapeDtypeStruct((N, F), jnp.bfloat16) for _ in range(n_bf)]
    )
    out_specs = tuple(full for _ in range(n_f32 + n_bf))
    scratch = [pltpu.VMEM((N, F), jnp.float32)]
    if need_r:
        scratch.append(pltpu.VMEM((N, F), jnp.float32))

    return pl.pallas_call(
        _make_hop_body(gi, gj, BI, BJ, need_r, need_next, has_acc, relu, bf_out),
        grid=(gi, gj),
        in_specs=in_specs,
        out_specs=out_specs,
        out_shape=out_shape,
        scratch_shapes=scratch,
    )(*ops)


def kernel(l_feat, r_feat, network, W1a, b1a, W2a, b2a, W1b, b1b, W2b, b2b):
    A16 = network.astype(jnp.bfloat16)
    lbf = l_feat.astype(jnp.bfloat16)
    rbf = r_feat.astype(jnp.bfloat16)
    b1a = b1a.reshape(1, F)
    b2a = b2a.reshape(1, F)
    b1b = b1b.reshape(1, F)
    b2b = b2b.reshape(1, F)

    ol, orv, nl, nr, nlbf, nrbf = _hop(
        A16, rbf, lbf, l_feat, r_feat, None, W1a, b1a, W2a, b2a,
        need_r=True, need_next=True, relu=False, bf_out=False,
    )
    y1, z1, y1bf, z1bf = _hop(
        A16, nrbf, nlbf, nl, nr, (ol, orv), W1a, b1a, W2a, b2a,
        need_r=True, need_next=False, relu=True, bf_out=True,
    )
    ol2, or2, nl2, nr2, nl2bf, nr2bf = _hop(
        A16, z1bf, y1bf, y1, z1, None, W1b, b1b, W2b, b2b,
        need_r=True, need_next=True, relu=False, bf_out=False,
    )
    (y2,) = _hop(
        A16, nr2bf, None, nl2, None, (ol2,), W1b, b1b, W2b, b2b,
        need_r=False, need_next=False, relu=True, bf_out=False,
    )
    return y2


# speedup vs baseline: 1.4984x; 1.0995x over previous
import jax
import jax.numpy as jnp
from jax.experimental import pallas as pl
from jax.experimental.pallas import tpu as pltpu

N = 4096
F = 128
BI = 1024


def _make_hop_body(gi, bi, need_r, need_next, has_acc, relu, bf_out):
    def body(*refs):
        it = iter(refs)
        A = next(it)
        x0b = next(it)
        x1s = next(it) if need_r else None
        x1f = next(it)
        x0f = next(it) if need_r else None
        accl = next(it) if has_acc else None
        accr = next(it) if (has_acc and need_r) else None
        W1 = next(it)
        b1 = next(it)
        W2 = next(it)
        b2 = next(it)
        outl = next(it)
        outr = next(it) if need_r else None
        nl = next(it) if need_next else None
        nr = next(it) if need_next else None
        nlbf = next(it) if (need_next or bf_out) else None
        nrbf = next(it) if (need_next or bf_out) else None
        yr = next(it) if need_r else None

        i = pl.program_id(0)
        a = A[...]

        W1v = W1[...].astype(jnp.bfloat16)
        W2v = W2[...].astype(jnp.bfloat16)
        bias = b1[...] + b2[...]

        ylv = jax.lax.dot_general(
            a, x0b[...], (((1,), (0,)), ((), ())),
            preferred_element_type=jnp.float32,
        )
        lm = ylv * x1f[...]
        ol = (
            jnp.dot(ylv.astype(jnp.bfloat16), W1v, preferred_element_type=jnp.float32)
            + jnp.dot(lm.astype(jnp.bfloat16), W2v, preferred_element_type=jnp.float32)
            + bias
        )
        if has_acc:
            ol = ol + accl[...]
        if relu:
            ol = jnp.maximum(ol, 0.0)
        outl[...] = ol
        if need_next:
            nx = ylv + lm
            nl[...] = nx
            nlbf[...] = nx.astype(jnp.bfloat16)
        if bf_out:
            nlbf[...] = ol.astype(jnp.bfloat16)

        if need_r:
            part_r = jax.lax.dot_general(
                a, x1s[...], (((0,), (0,)), ((), ())),
                preferred_element_type=jnp.float32,
            )

            @pl.when(i == 0)
            def _():
                yr[...] = part_r

            @pl.when(i != 0)
            def _():
                yr[...] += part_r

            @pl.when(i == gi - 1)
            def _():
                yrv = yr[...]
                rm = yrv * x0f[...]
                orv = (
                    jnp.dot(yrv.astype(jnp.bfloat16), W1v, preferred_element_type=jnp.float32)
                    + jnp.dot(rm.astype(jnp.bfloat16), W2v, preferred_element_type=jnp.float32)
                    + bias
                )
                if has_acc:
                    orv = orv + accr[...]
                if relu:
                    orv = jnp.maximum(orv, 0.0)
                outr[...] = orv
                if need_next:
                    nx = yrv + rm
                    nr[...] = nx
                    nrbf[...] = nx.astype(jnp.bfloat16)
                if bf_out:
                    nrbf[...] = orv.astype(jnp.bfloat16)

    return body


def _hop(A16, x0bf, x1bf, x1f, x0f, accs, W1, b1, W2, b2,
         *, need_r, need_next, relu, bf_out):
    has_acc = accs is not None
    gi = N // BI
    full = pl.BlockSpec((N, F), lambda i: (0, 0))
    strip = pl.BlockSpec((BI, F), lambda i: (i, 0))
    wspec = pl.BlockSpec((F, F), lambda i: (0, 0))
    bspec = pl.BlockSpec((1, F), lambda i: (0, 0))
    in_specs = [
        pl.BlockSpec((BI, N), lambda i: (i, 0)),
        full,
    ]
    ops = [A16, x0bf]
    if need_r:
        in_specs.append(strip)
        ops.append(x1bf)
    in_specs.append(strip)
    ops.append(x1f)
    if need_r:
        in_specs.append(full)
        ops.append(x0f)
    if has_acc:
        in_specs.append(strip)
        ops.append(accs[0])
        if need_r:
            in_specs.append(full)
            ops.append(accs[1])
    in_specs += [wspec, bspec, wspec, bspec]
    ops += [W1, b1, W2, b2]

    out_shape = [jax.ShapeDtypeStruct((N, F), jnp.float32)]
    out_specs = [strip]
    if need_r:
        out_shape.append(jax.ShapeDtypeStruct((N, F), jnp.float32))
        out_specs.append(full)
    if need_next:
        out_shape += [jax.ShapeDtypeStruct((N, F), jnp.float32)] * 2
        out_specs += [strip, full]
    if need_next or bf_out:
        out_shape += [jax.ShapeDtypeStruct((N, F), jnp.bfloat16)] * 2
        out_specs += [strip, full]
    scratch = [pltpu.VMEM((N, F), jnp.float32)] if need_r else []

    return pl.pallas_call(
        _make_hop_body(gi, BI, need_r, need_next, has_acc, relu, bf_out),
        grid=(gi,),
        in_specs=in_specs,
        out_specs=tuple(out_specs),
        out_shape=tuple(out_shape),
        scratch_shapes=scratch,
    )(*ops)


def kernel(l_feat, r_feat, network, W1a, b1a, W2a, b2a, W1b, b1b, W2b, b2b):
    A16 = network.astype(jnp.bfloat16)
    lbf = l_feat.astype(jnp.bfloat16)
    rbf = r_feat.astype(jnp.bfloat16)
    b1a = b1a.reshape(1, F)
    b2a = b2a.reshape(1, F)
    b1b = b1b.reshape(1, F)
    b2b = b2b.reshape(1, F)

    ol, orv, nl, nr, nlbf, nrbf = _hop(
        A16, rbf, lbf, l_feat, r_feat, None, W1a, b1a, W2a, b2a,
        need_r=True, need_next=True, relu=False, bf_out=False,
    )
    y1, z1, y1bf, z1bf = _hop(
        A16, nrbf, nlbf, nl, nr, (ol, orv), W1a, b1a, W2a, b2a,
        need_r=True, need_next=False, relu=True, bf_out=True,
    )
    ol2, or2, nl2, nr2, nl2bf, nr2bf = _hop(
        A16, z1bf, y1bf, y1, z1, None, W1b, b1b, W2b, b2b,
        need_r=True, need_next=True, relu=False, bf_out=False,
    )
    (y2,) = _hop(
        A16, nr2bf, None, nl2, None, (ol2,), W1b, b1b, W2b, b2b,
        need_r=False, need_next=False, relu=True, bf_out=False,
    )
    return y2


# speedup vs baseline: 1.7257x; 1.1517x over previous
import jax
import jax.numpy as jnp
from jax.experimental import pallas as pl
from jax.experimental.pallas import tpu as pltpu

N = 4096
F = 128
BI = 1024


def _make_hop_body(gi, bi, need_r, need_next, has_acc, relu, bf_out):
    def body(*refs):
        it = iter(refs)
        A = next(it)
        x0b = next(it)
        x1s = next(it) if need_r else None
        x1f = next(it)
        x0f = next(it) if need_r else None
        accl = next(it) if has_acc else None
        accr = next(it) if (has_acc and need_r) else None
        W1 = next(it)
        b1 = next(it)
        W2 = next(it)
        b2 = next(it)
        outl = next(it)
        outr = next(it) if need_r else None
        nl = next(it) if need_next else None
        nr = next(it) if need_next else None
        nlbf = next(it) if (need_next or bf_out) else None
        nrbf = next(it) if (need_next or bf_out) else None
        yr = next(it) if need_r else None

        i = pl.program_id(0)
        a = A[...]

        W1v = W1[...].astype(jnp.bfloat16)
        W2v = W2[...].astype(jnp.bfloat16)
        bias = b1[...] + b2[...]

        ylv = jax.lax.dot_general(
            a, x0b[...], (((1,), (0,)), ((), ())),
            preferred_element_type=jnp.float32,
        )
        lm = ylv * x1f[...]
        ol = (
            jnp.dot(ylv.astype(jnp.bfloat16), W1v, preferred_element_type=jnp.float32)
            + jnp.dot(lm.astype(jnp.bfloat16), W2v, preferred_element_type=jnp.float32)
            + bias
        )
        if has_acc:
            ol = ol + accl[...]
        if relu:
            ol = jnp.maximum(ol, 0.0)
        outl[...] = ol
        if need_next:
            nx = ylv + lm
            nl[...] = nx
            nlbf[...] = nx.astype(jnp.bfloat16)
        if bf_out:
            nlbf[...] = ol.astype(jnp.bfloat16)

        if need_r:
            x1v = x1s[...]
            cj = 1024 if N % 1024 == 0 else N
            for jj in range(N // cj):
                sl = slice(jj * cj, (jj + 1) * cj)
                pT = jax.lax.dot_general(
                    x1v, a[:, sl], (((0,), (0,)), ((), ())),
                    preferred_element_type=jnp.float32,
                )

                @pl.when(i == 0)
                def _():
                    yr[:, sl] = pT

                @pl.when(i != 0)
                def _():
                    yr[:, sl] += pT

            @pl.when(i == gi - 1)
            def _():
                yrv = jnp.transpose(yr[...])
                rm = yrv * x0f[...]
                orv = (
                    jnp.dot(yrv.astype(jnp.bfloat16), W1v, preferred_element_type=jnp.float32)
                    + jnp.dot(rm.astype(jnp.bfloat16), W2v, preferred_element_type=jnp.float32)
                    + bias
                )
                if has_acc:
                    orv = orv + accr[...]
                if relu:
                    orv = jnp.maximum(orv, 0.0)
                outr[...] = orv
                if need_next:
                    nx = yrv + rm
                    nr[...] = nx
                    nrbf[...] = nx.astype(jnp.bfloat16)
                if bf_out:
                    nrbf[...] = orv.astype(jnp.bfloat16)

    return body


def _hop(A16, x0bf, x1bf, x1f, x0f, accs, W1, b1, W2, b2,
         *, need_r, need_next, relu, bf_out):
    has_acc = accs is not None
    gi = N // BI
    full = pl.BlockSpec((N, F), lambda i: (0, 0))
    strip = pl.BlockSpec((BI, F), lambda i: (i, 0))
    wspec = pl.BlockSpec((F, F), lambda i: (0, 0))
    bspec = pl.BlockSpec((1, F), lambda i: (0, 0))
    in_specs = [
        pl.BlockSpec((BI, N), lambda i: (i, 0)),
        full,
    ]
    ops = [A16, x0bf]
    if need_r:
        in_specs.append(strip)
        ops.append(x1bf)
    in_specs.append(strip)
    ops.append(x1f)
    if need_r:
        in_specs.append(full)
        ops.append(x0f)
    if has_acc:
        in_specs.append(strip)
        ops.append(accs[0])
        if need_r:
            in_specs.append(full)
            ops.append(accs[1])
    in_specs += [wspec, bspec, wspec, bspec]
    ops += [W1, b1, W2, b2]

    out_shape = [jax.ShapeDtypeStruct((N, F), jnp.float32)]
    out_specs = [strip]
    if need_r:
        out_shape.append(jax.ShapeDtypeStruct((N, F), jnp.float32))
        out_specs.append(full)
    if need_next:
        out_shape += [jax.ShapeDtypeStruct((N, F), jnp.float32)] * 2
        out_specs += [strip, full]
    if need_next or bf_out:
        out_shape += [jax.ShapeDtypeStruct((N, F), jnp.bfloat16)] * 2
        out_specs += [strip, full]
    scratch = [pltpu.VMEM((F, N), jnp.float32)] if need_r else []

    return pl.pallas_call(
        _make_hop_body(gi, BI, need_r, need_next, has_acc, relu, bf_out),
        grid=(gi,),
        in_specs=in_specs,
        out_specs=tuple(out_specs),
        out_shape=tuple(out_shape),
        scratch_shapes=scratch,
    )(*ops)


def kernel(l_feat, r_feat, network, W1a, b1a, W2a, b2a, W1b, b1b, W2b, b2b):
    A16 = network.astype(jnp.bfloat16)
    lbf = l_feat.astype(jnp.bfloat16)
    rbf = r_feat.astype(jnp.bfloat16)
    b1a = b1a.reshape(1, F)
    b2a = b2a.reshape(1, F)
    b1b = b1b.reshape(1, F)
    b2b = b2b.reshape(1, F)

    ol, orv, nl, nr, nlbf, nrbf = _hop(
        A16, rbf, lbf, l_feat, r_feat, None, W1a, b1a, W2a, b2a,
        need_r=True, need_next=True, relu=False, bf_out=False,
    )
    y1, z1, y1bf, z1bf = _hop(
        A16, nrbf, nlbf, nl, nr, (ol, orv), W1a, b1a, W2a, b2a,
        need_r=True, need_next=False, relu=True, bf_out=True,
    )
    ol2, or2, nl2, nr2, nl2bf, nr2bf = _hop(
        A16, z1bf, y1bf, y1, z1, None, W1b, b1b, W2b, b2b,
        need_r=True, need_next=True, relu=False, bf_out=False,
    )
    (y2,) = _hop(
        A16, nr2bf, None, nl2, None, (ol2,), W1b, b1b, W2b, b2b,
        need_r=False, need_next=False, relu=True, bf_out=False,
    )
    return y2


# speedup vs baseline: 1.9886x; 1.1524x over previous
import jax
import jax.numpy as jnp
from jax.experimental import pallas as pl
from jax.experimental.pallas import tpu as pltpu

N = 4096
F = 128
BI = 1024
BI_CAST = 512


def _make_hop_body(gi, bi, need_r, need_next, has_acc, relu, bf_out, cast_a):
    def body(*refs):
        it = iter(refs)
        A = next(it)
        x0b = next(it)
        x1s = next(it) if need_r else None
        x1f = next(it)
        x0f = next(it) if need_r else None
        accl = next(it) if has_acc else None
        accr = next(it) if (has_acc and need_r) else None
        W1 = next(it)
        b1 = next(it)
        W2 = next(it)
        b2 = next(it)
        a16out = next(it) if cast_a else None
        outl = next(it)
        outr = next(it) if need_r else None
        nl = next(it) if need_next else None
        nr = next(it) if need_next else None
        nlbf = next(it) if (need_next or bf_out) else None
        nrbf = next(it) if (need_next or bf_out) else None
        yr = next(it) if need_r else None

        i = pl.program_id(0)
        if cast_a:
            a = A[...].astype(jnp.bfloat16)
            a16out[...] = a
        else:
            a = A[...]

        W1v = W1[...].astype(jnp.bfloat16)
        W2v = W2[...].astype(jnp.bfloat16)
        bias = b1[...] + b2[...]

        ylv = jax.lax.dot_general(
            a, x0b[...], (((1,), (0,)), ((), ())),
            preferred_element_type=jnp.float32,
        )
        lm = ylv * x1f[...]
        ol = (
            jnp.dot(ylv.astype(jnp.bfloat16), W1v, preferred_element_type=jnp.float32)
            + jnp.dot(lm.astype(jnp.bfloat16), W2v, preferred_element_type=jnp.float32)
            + bias
        )
        if has_acc:
            ol = ol + accl[...]
        if relu:
            ol = jnp.maximum(ol, 0.0)
        outl[...] = ol
        if need_next:
            nx = ylv + lm
            nl[...] = nx
            nlbf[...] = nx.astype(jnp.bfloat16)
        if bf_out:
            nlbf[...] = ol.astype(jnp.bfloat16)

        if need_r:
            x1v = x1s[...]
            cj = 1024 if N % 1024 == 0 else N
            for jj in range(N // cj):
                sl = slice(jj * cj, (jj + 1) * cj)
                pT = jax.lax.dot_general(
                    x1v, a[:, sl], (((0,), (0,)), ((), ())),
                    preferred_element_type=jnp.float32,
                )

                @pl.when(i == 0)
                def _():
                    yr[:, sl] = pT

                @pl.when(i != 0)
                def _():
                    yr[:, sl] += pT

            @pl.when(i == gi - 1)
            def _():
                yrv = jnp.transpose(yr[...])
                rm = yrv * x0f[...]
                orv = (
                    jnp.dot(yrv.astype(jnp.bfloat16), W1v, preferred_element_type=jnp.float32)
                    + jnp.dot(rm.astype(jnp.bfloat16), W2v, preferred_element_type=jnp.float32)
                    + bias
                )
                if has_acc:
                    orv = orv + accr[...]
                if relu:
                    orv = jnp.maximum(orv, 0.0)
                outr[...] = orv
                if need_next:
                    nx = yrv + rm
                    nr[...] = nx
                    nrbf[...] = nx.astype(jnp.bfloat16)
                if bf_out:
                    nrbf[...] = orv.astype(jnp.bfloat16)

    return body


def _hop(A_in, x0bf, x1bf, x1f, x0f, accs, W1, b1, W2, b2,
         *, need_r, need_next, relu, bf_out, cast_a=False):
    has_acc = accs is not None
    bi = BI_CAST if cast_a else BI
    gi = N // bi
    full = pl.BlockSpec((N, F), lambda i: (0, 0))
    strip = pl.BlockSpec((bi, F), lambda i: (i, 0))
    wspec = pl.BlockSpec((F, F), lambda i: (0, 0))
    bspec = pl.BlockSpec((1, F), lambda i: (0, 0))
    in_specs = [
        pl.BlockSpec((bi, N), lambda i: (i, 0)),
        full,
    ]
    ops = [A_in, x0bf]
    if need_r:
        in_specs.append(strip)
        ops.append(x1bf)
    in_specs.append(strip)
    ops.append(x1f)
    if need_r:
        in_specs.append(full)
        ops.append(x0f)
    if has_acc:
        in_specs.append(strip)
        ops.append(accs[0])
        if need_r:
            in_specs.append(full)
            ops.append(accs[1])
    in_specs += [wspec, bspec, wspec, bspec]
    ops += [W1, b1, W2, b2]

    out_shape = []
    out_specs = []
    if cast_a:
        out_shape.append(jax.ShapeDtypeStruct((N, N), jnp.bfloat16))
        out_specs.append(pl.BlockSpec((bi, N), lambda i: (i, 0)))
    out_shape.append(jax.ShapeDtypeStruct((N, F), jnp.float32))
    out_specs.append(strip)
    if need_r:
        out_shape.append(jax.ShapeDtypeStruct((N, F), jnp.float32))
        out_specs.append(full)
    if need_next:
        out_shape += [jax.ShapeDtypeStruct((N, F), jnp.float32)] * 2
        out_specs += [strip, full]
    if need_next or bf_out:
        out_shape += [jax.ShapeDtypeStruct((N, F), jnp.bfloat16)] * 2
        out_specs += [strip, full]
    scratch = [pltpu.VMEM((F, N), jnp.float32)] if need_r else []

    return pl.pallas_call(
        _make_hop_body(gi, bi, need_r, need_next, has_acc, relu, bf_out, cast_a),
        grid=(gi,),
        in_specs=in_specs,
        out_specs=tuple(out_specs),
        out_shape=tuple(out_shape),
        scratch_shapes=scratch,
    )(*ops)


def kernel(l_feat, r_feat, network, W1a, b1a, W2a, b2a, W1b, b1b, W2b, b2b):
    lbf = l_feat.astype(jnp.bfloat16)
    rbf = r_feat.astype(jnp.bfloat16)
    b1a = b1a.reshape(1, F)
    b2a = b2a.reshape(1, F)
    b1b = b1b.reshape(1, F)
    b2b = b2b.reshape(1, F)

    A16, ol, orv, nl, nr, nlbf, nrbf = _hop(
        network, rbf, lbf, l_feat, r_feat, None, W1a, b1a, W2a, b2a,
        need_r=True, need_next=True, relu=False, bf_out=False, cast_a=True,
    )
    y1, z1, y1bf, z1bf = _hop(
        A16, nrbf, nlbf, nl, nr, (ol, orv), W1a, b1a, W2a, b2a,
        need_r=True, need_next=False, relu=True, bf_out=True,
    )
    ol2, or2, nl2, nr2, nl2bf, nr2bf = _hop(
        A16, z1bf, y1bf, y1, z1, None, W1b, b1b, W2b, b2b,
        need_r=True, need_next=True, relu=False, bf_out=False,
    )
    (y2,) = _hop(
        A16, nr2bf, None, nl2, None, (ol2,), W1b, b1b, W2b, b2b,
        need_r=False, need_next=False, relu=True, bf_out=False,
    )
    return y2


# speedup vs baseline: 2.1909x; 1.1017x over previous
import jax
import jax.numpy as jnp
from jax.experimental import pallas as pl
from jax.experimental.pallas import tpu as pltpu

N = 4096
F = 128
BI = 1024
BI_CAST = 512


def _make_hop_body(gi, bi, need_r, need_next, has_acc, relu, bf_out, cast_a):
    def body(*refs):
        it = iter(refs)
        A = next(it)
        x0b = next(it)
        x1s = next(it) if need_r else None
        x1f = next(it)
        x0f = next(it) if need_r else None
        accl = next(it) if has_acc else None
        accr = next(it) if (has_acc and need_r) else None
        W1 = next(it)
        b1 = next(it)
        W2 = next(it)
        b2 = next(it)
        a16out = next(it) if cast_a else None
        outl = next(it)
        outr = next(it) if need_r else None
        nl = next(it) if need_next else None
        nr = next(it) if need_next else None
        nlbf = next(it) if (need_next or bf_out) else None
        nrbf = next(it) if (need_next or bf_out) else None
        yr = next(it) if need_r else None

        i = pl.program_id(0)
        if cast_a:
            a = A[...].astype(jnp.bfloat16)
            a16out[...] = a
        else:
            a = A[...]

        W1v = W1[...].astype(jnp.bfloat16)
        W2v = W2[...].astype(jnp.bfloat16)
        bias = b1[...] + b2[...]

        ylv = jax.lax.dot_general(
            a, x0b[...], (((1,), (0,)), ((), ())),
            preferred_element_type=jnp.float32,
        )
        lm = ylv * x1f[...]
        ol = (
            jnp.dot(ylv.astype(jnp.bfloat16), W1v, preferred_element_type=jnp.float32)
            + jnp.dot(lm.astype(jnp.bfloat16), W2v, preferred_element_type=jnp.float32)
            + bias
        )
        if has_acc:
            ol = ol + accl[...]
        if relu:
            ol = jnp.maximum(ol, 0.0)
        outl[...] = ol
        if need_next:
            nx = ylv + lm
            nl[...] = nx
            nlbf[...] = nx.astype(jnp.bfloat16)
        if bf_out:
            nlbf[...] = ol.astype(jnp.bfloat16)

        if need_r:
            x1v = x1s[...]
            cj = 1024 if N % 1024 == 0 else N
            for jj in range(N // cj):
                sl = slice(jj * cj, (jj + 1) * cj)
                pT = jax.lax.dot_general(
                    x1v, a[:, sl], (((0,), (0,)), ((), ())),
                    preferred_element_type=jnp.float32,
                )

                @pl.when(i == 0)
                def _():
                    yr[:, sl] = pT

                @pl.when(i != 0)
                def _():
                    yr[:, sl] += pT

            @pl.when(i == gi - 1)
            def _():
                yrv = jnp.transpose(yr[...])
                rm = yrv * x0f[...]
                orv = (
                    jnp.dot(yrv.astype(jnp.bfloat16), W1v, preferred_element_type=jnp.float32)
                    + jnp.dot(rm.astype(jnp.bfloat16), W2v, preferred_element_type=jnp.float32)
                    + bias
                )
                if has_acc:
                    orv = orv + accr[...]
                if relu:
                    orv = jnp.maximum(orv, 0.0)
                outr[...] = orv
                if need_next:
                    nx = yrv + rm
                    nr[...] = nx
                    nrbf[...] = nx.astype(jnp.bfloat16)
                if bf_out:
                    nrbf[...] = orv.astype(jnp.bfloat16)

    return body


def _hop(A_in, x0bf, x1bf, x1f, x0f, accs, W1, b1, W2, b2,
         *, need_r, need_next, relu, bf_out, cast_a=False):
    has_acc = accs is not None
    bi = BI_CAST if cast_a else BI
    gi = N // bi
    full = pl.BlockSpec((N, F), lambda i: (0, 0))
    strip = pl.BlockSpec((bi, F), lambda i: (i, 0))
    wspec = pl.BlockSpec((F, F), lambda i: (0, 0))
    bspec = pl.BlockSpec((1, F), lambda i: (0, 0))
    in_specs = [
        pl.BlockSpec((bi, N), lambda i: (i, 0)),
        full,
    ]
    ops = [A_in, x0bf]
    if need_r:
        in_specs.append(strip)
        ops.append(x1bf)
    in_specs.append(strip)
    ops.append(x1f)
    if need_r:
        in_specs.append(full)
        ops.append(x0f)
    if has_acc:
        in_specs.append(strip)
        ops.append(accs[0])
        if need_r:
            in_specs.append(full)
            ops.append(accs[1])
    in_specs += [wspec, bspec, wspec, bspec]
    ops += [W1, b1, W2, b2]

    out_shape = []
    out_specs = []
    if cast_a:
        out_shape.append(jax.ShapeDtypeStruct((N, N), jnp.bfloat16))
        out_specs.append(pl.BlockSpec((bi, N), lambda i: (i, 0)))
    out_shape.append(jax.ShapeDtypeStruct((N, F), jnp.float32))
    out_specs.append(strip)
    if need_r:
        out_shape.append(jax.ShapeDtypeStruct((N, F), jnp.float32))
        out_specs.append(full)
    if need_next:
        out_shape += [jax.ShapeDtypeStruct((N, F), jnp.float32)] * 2
        out_specs += [strip, full]
    if need_next or bf_out:
        out_shape += [jax.ShapeDtypeStruct((N, F), jnp.bfloat16)] * 2
        out_specs += [strip, full]
    scratch = [pltpu.VMEM((F, N), jnp.float32)] if need_r else []

    return pl.pallas_call(
        _make_hop_body(gi, bi, need_r, need_next, has_acc, relu, bf_out, cast_a),
        grid=(gi,),
        in_specs=in_specs,
        out_specs=tuple(out_specs),
        out_shape=tuple(out_shape),
        scratch_shapes=scratch,
    )(*ops)


def _mega_body(gi, bi):
    def body(A16, ol0, or0, nl0, nr0, nlbf0, nrbf0,
             W1a, b1a, W2a, b2a, W1b, b1b, W2b, b2b,
             y2out,
             yrT, y1, z1, ol2, nl2, y1bf, z1bf, nr2bf):
        p = pl.program_id(0)
        i = pl.program_id(1)
        a = A16[...]
        sl_i = pl.ds(i * bi, bi)
        cj = 1024 if N % 1024 == 0 else N

        def rdot_accum(x1v):
            for jj in range(N // cj):
                sl = slice(jj * cj, (jj + 1) * cj)
                pT = jax.lax.dot_general(
                    x1v, a[:, sl], (((0,), (0,)), ((), ())),
                    preferred_element_type=jnp.float32,
                )

                @pl.when(i == 0)
                def _():
                    yrT[:, sl] = pT

                @pl.when(i != 0)
                def _():
                    yrT[:, sl] += pT

        @pl.when(p == 0)
        def _():
            W1v = W1a[...].astype(jnp.bfloat16)
            W2v = W2a[...].astype(jnp.bfloat16)
            bias = b1a[...] + b2a[...]
            ylv = jax.lax.dot_general(
                a, nrbf0[...], (((1,), (0,)), ((), ())),
                preferred_element_type=jnp.float32,
            )
            lm = ylv * nl0[sl_i, :]
            olv = (
                jnp.dot(ylv.astype(jnp.bfloat16), W1v, preferred_element_type=jnp.float32)
                + jnp.dot(lm.astype(jnp.bfloat16), W2v, preferred_element_type=jnp.float32)
                + bias + ol0[sl_i, :]
            )
            y1v = jnp.maximum(olv, 0.0)
            y1[sl_i, :] = y1v
            y1bf[sl_i, :] = y1v.astype(jnp.bfloat16)
            rdot_accum(nlbf0[sl_i, :])

            @pl.when(i == gi - 1)
            def _():
                yrv = jnp.transpose(yrT[...])
                rm = yrv * nr0[...]
                orv = (
                    jnp.dot(yrv.astype(jnp.bfloat16), W1v, preferred_element_type=jnp.float32)
                    + jnp.dot(rm.astype(jnp.bfloat16), W2v, preferred_element_type=jnp.float32)
                    + bias + or0[...]
                )
                z1v = jnp.maximum(orv, 0.0)
                z1[...] = z1v
                z1bf[...] = z1v.astype(jnp.bfloat16)

        @pl.when(p == 1)
        def _():
            W1v = W1b[...].astype(jnp.bfloat16)
            W2v = W2b[...].astype(jnp.bfloat16)
            bias = b1b[...] + b2b[...]
            ylv = jax.lax.dot_general(
                a, z1bf[...], (((1,), (0,)), ((), ())),
                preferred_element_type=jnp.float32,
            )
            lm = ylv * y1[sl_i, :]
            ol2[sl_i, :] = (
                jnp.dot(ylv.astype(jnp.bfloat16), W1v, preferred_element_type=jnp.float32)
                + jnp.dot(lm.astype(jnp.bfloat16), W2v, preferred_element_type=jnp.float32)
                + bias
            )
            nl2[sl_i, :] = ylv + lm
            rdot_accum(y1bf[sl_i, :])

            @pl.when(i == gi - 1)
            def _():
                yrv = jnp.transpose(yrT[...])
                nr2 = yrv + yrv * z1[...]
                nr2bf[...] = nr2.astype(jnp.bfloat16)

        @pl.when(p == 2)
        def _():
            W1v = W1b[...].astype(jnp.bfloat16)
            W2v = W2b[...].astype(jnp.bfloat16)
            bias = b1b[...] + b2b[...]
            ylv = jax.lax.dot_general(
                a, nr2bf[...], (((1,), (0,)), ((), ())),
                preferred_element_type=jnp.float32,
            )
            lm = ylv * nl2[sl_i, :]
            y2v = (
                jnp.dot(ylv.astype(jnp.bfloat16), W1v, preferred_element_type=jnp.float32)
                + jnp.dot(lm.astype(jnp.bfloat16), W2v, preferred_element_type=jnp.float32)
                + bias + ol2[sl_i, :]
            )
            y2out[...] = jnp.maximum(y2v, 0.0)

    return body


def _mega(A16, ol0, or0, nl0, nr0, nlbf0, nrbf0,
          W1a, b1a, W2a, b2a, W1b, b1b, W2b, b2b):
    gi = N // BI
    a_spec = pl.BlockSpec((BI, N), lambda p, i: (i, 0))
    full = pl.BlockSpec((N, F), lambda p, i: (0, 0))
    wspec = pl.BlockSpec((F, F), lambda p, i: (0, 0))
    bspec = pl.BlockSpec((1, F), lambda p, i: (0, 0))
    in_specs = [a_spec] + [full] * 6 + [wspec, bspec, wspec, bspec] * 2
    scratch = [
        pltpu.VMEM((F, N), jnp.float32),
        pltpu.VMEM((N, F), jnp.float32),
        pltpu.VMEM((N, F), jnp.float32),
        pltpu.VMEM((N, F), jnp.float32),
        pltpu.VMEM((N, F), jnp.float32),
        pltpu.VMEM((N, F), jnp.bfloat16),
        pltpu.VMEM((N, F), jnp.bfloat16),
        pltpu.VMEM((N, F), jnp.bfloat16),
    ]
    return pl.pallas_call(
        _mega_body(gi, BI),
        grid=(3, gi),
        in_specs=in_specs,
        out_specs=pl.BlockSpec((BI, F), lambda p, i: (i, 0)),
        out_shape=jax.ShapeDtypeStruct((N, F), jnp.float32),
        scratch_shapes=scratch,
    )(A16, ol0, or0, nl0, nr0, nlbf0, nrbf0,
      W1a, b1a, W2a, b2a, W1b, b1b, W2b, b2b)


def kernel(l_feat, r_feat, network, W1a, b1a, W2a, b2a, W1b, b1b, W2b, b2b):
    lbf = l_feat.astype(jnp.bfloat16)
    rbf = r_feat.astype(jnp.bfloat16)
    b1a = b1a.reshape(1, F)
    b2a = b2a.reshape(1, F)
    b1b = b1b.reshape(1, F)
    b2b = b2b.reshape(1, F)

    A16, ol, orv, nl, nr, nlbf, nrbf = _hop(
        network, rbf, lbf, l_feat, r_feat, None, W1a, b1a, W2a, b2a,
        need_r=True, need_next=True, relu=False, bf_out=False, cast_a=True,
    )
    return _mega(
        A16, ol, orv, nl, nr, nlbf, nrbf,
        W1a, b1a, W2a, b2a, W1b, b1b, W2b, b2b,
    )


# speedup vs baseline: 2.2946x; 1.0473x over previous
import jax
import jax.numpy as jnp
from jax.experimental import pallas as pl
from jax.experimental.pallas import tpu as pltpu

N = 4096
F = 128
BI = 1024
BI_CAST = 512


def _make_hop_body(gi, bi, need_r, need_next, has_acc, relu, bf_out, cast_a):
    def body(*refs):
        it = iter(refs)
        A = next(it)
        x0b = next(it)
        x1s = next(it) if need_r else None
        x1f = next(it)
        x0f = next(it) if need_r else None
        accl = next(it) if has_acc else None
        accr = next(it) if (has_acc and need_r) else None
        W1 = next(it)
        b1 = next(it)
        W2 = next(it)
        b2 = next(it)
        a16out = next(it) if cast_a else None
        outl = next(it)
        outr = next(it) if need_r else None
        nl = next(it) if need_next else None
        nr = next(it) if need_next else None
        nlbf = next(it) if (need_next or bf_out) else None
        nrbf = next(it) if (need_next or bf_out) else None
        yr = next(it) if need_r else None

        i = pl.program_id(0)
        if cast_a:
            a = A[...].astype(jnp.bfloat16)
            a16out[...] = a
        else:
            a = A[...]

        W1v = W1[...].astype(jnp.bfloat16)
        W2v = W2[...].astype(jnp.bfloat16)
        bias = b1[...] + b2[...]

        ylv = jax.lax.dot_general(
            a, x0b[...], (((1,), (0,)), ((), ())),
            preferred_element_type=jnp.float32,
        )
        lm = ylv * x1f[...]
        ol = (
            jnp.dot(ylv.astype(jnp.bfloat16), W1v, preferred_element_type=jnp.float32)
            + jnp.dot(lm.astype(jnp.bfloat16), W2v, preferred_element_type=jnp.float32)
            + bias
        )
        if has_acc:
            ol = ol + accl[...]
        if relu:
            ol = jnp.maximum(ol, 0.0)
        outl[...] = ol
        if need_next:
            nx = ylv + lm
            nl[...] = nx
            nlbf[...] = nx.astype(jnp.bfloat16)
        if bf_out:
            nlbf[...] = ol.astype(jnp.bfloat16)

        if need_r:
            x1v = x1s[...]
            cj = 2048 if N % 2048 == 0 else N
            for jj in range(N // cj):
                sl = slice(jj * cj, (jj + 1) * cj)
                pT = jax.lax.dot_general(
                    x1v, a[:, sl], (((0,), (0,)), ((), ())),
                    preferred_element_type=jnp.float32,
                )

                @pl.when(i == 0)
                def _():
                    yr[:, sl] = pT

                @pl.when(i != 0)
                def _():
                    yr[:, sl] += pT

            @pl.when(i == gi - 1)
            def _():
                yrv = jnp.transpose(yr[...])
                rm = yrv * x0f[...]
                orv = (
                    jnp.dot(yrv.astype(jnp.bfloat16), W1v, preferred_element_type=jnp.float32)
                    + jnp.dot(rm.astype(jnp.bfloat16), W2v, preferred_element_type=jnp.float32)
                    + bias
                )
                if has_acc:
                    orv = orv + accr[...]
                if relu:
                    orv = jnp.maximum(orv, 0.0)
                outr[...] = orv
                if need_next:
                    nx = yrv + rm
                    nr[...] = nx
                    nrbf[...] = nx.astype(jnp.bfloat16)
                if bf_out:
                    nrbf[...] = orv.astype(jnp.bfloat16)

    return body


def _hop(A_in, x0bf, x1bf, x1f, x0f, accs, W1, b1, W2, b2,
         *, need_r, need_next, relu, bf_out, cast_a=False):
    has_acc = accs is not None
    bi = BI_CAST if cast_a else BI
    gi = N // bi
    full = pl.BlockSpec((N, F), lambda i: (0, 0))
    strip = pl.BlockSpec((bi, F), lambda i: (i, 0))
    wspec = pl.BlockSpec((F, F), lambda i: (0, 0))
    bspec = pl.BlockSpec((1, F), lambda i: (0, 0))
    in_specs = [
        pl.BlockSpec((bi, N), lambda i: (i, 0)),
        full,
    ]
    ops = [A_in, x0bf]
    if need_r:
        in_specs.append(strip)
        ops.append(x1bf)
    in_specs.append(strip)
    ops.append(x1f)
    if need_r:
        in_specs.append(full)
        ops.append(x0f)
    if has_acc:
        in_specs.append(strip)
        ops.append(accs[0])
        if need_r:
            in_specs.append(full)
            ops.append(accs[1])
    in_specs += [wspec, bspec, wspec, bspec]
    ops += [W1, b1, W2, b2]

    out_shape = []
    out_specs = []
    if cast_a:
        out_shape.append(jax.ShapeDtypeStruct((N, N), jnp.bfloat16))
        out_specs.append(pl.BlockSpec((bi, N), lambda i: (i, 0)))
    out_shape.append(jax.ShapeDtypeStruct((N, F), jnp.float32))
    out_specs.append(strip)
    if need_r:
        out_shape.append(jax.ShapeDtypeStruct((N, F), jnp.float32))
        out_specs.append(full)
    if need_next:
        out_shape += [jax.ShapeDtypeStruct((N, F), jnp.float32)] * 2
        out_specs += [strip, full]
    if need_next or bf_out:
        out_shape += [jax.ShapeDtypeStruct((N, F), jnp.bfloat16)] * 2
        out_specs += [strip, full]
    scratch = [pltpu.VMEM((F, N), jnp.float32)] if need_r else []

    return pl.pallas_call(
        _make_hop_body(gi, bi, need_r, need_next, has_acc, relu, bf_out, cast_a),
        grid=(gi,),
        in_specs=in_specs,
        out_specs=tuple(out_specs),
        out_shape=tuple(out_shape),
        scratch_shapes=scratch,
    )(*ops)


def _mega_body(gi, bi):
    def body(A16, ol0, or0, nl0, nr0, nlbf0, nrbf0,
             W1a, b1a, W2a, b2a, W1b, b1b, W2b, b2b,
             y2out,
             yrT, y1, z1, ol2, nl2, y1bf, z1bf, nr2bf):
        p = pl.program_id(0)
        i = pl.program_id(1)
        a = A16[...]
        sl_i = pl.ds(i * bi, bi)
        cj = 2048 if N % 2048 == 0 else N

        def rdot_accum(x1v):
            for jj in range(N // cj):
                sl = slice(jj * cj, (jj + 1) * cj)
                pT = jax.lax.dot_general(
                    x1v, a[:, sl], (((0,), (0,)), ((), ())),
                    preferred_element_type=jnp.float32,
                )

                @pl.when(i == 0)
                def _():
                    yrT[:, sl] = pT

                @pl.when(i != 0)
                def _():
                    yrT[:, sl] += pT

        @pl.when(p == 0)
        def _():
            W1v = W1a[...].astype(jnp.bfloat16)
            W2v = W2a[...].astype(jnp.bfloat16)
            bias = b1a[...] + b2a[...]
            ylv = jax.lax.dot_general(
                a, nrbf0[...], (((1,), (0,)), ((), ())),
                preferred_element_type=jnp.float32,
            )
            lm = ylv * nl0[sl_i, :]
            olv = (
                jnp.dot(ylv.astype(jnp.bfloat16), W1v, preferred_element_type=jnp.float32)
                + jnp.dot(lm.astype(jnp.bfloat16), W2v, preferred_element_type=jnp.float32)
                + bias + ol0[sl_i, :]
            )
            y1v = jnp.maximum(olv, 0.0)
            y1[sl_i, :] = y1v
            y1bf[sl_i, :] = y1v.astype(jnp.bfloat16)
            rdot_accum(nlbf0[sl_i, :])

            @pl.when(i == gi - 1)
            def _():
                yrv = jnp.transpose(yrT[...])
                rm = yrv * nr0[...]
                orv = (
                    jnp.dot(yrv.astype(jnp.bfloat16), W1v, preferred_element_type=jnp.float32)
                    + jnp.dot(rm.astype(jnp.bfloat16), W2v, preferred_element_type=jnp.float32)
                    + bias + or0[...]
                )
                z1v = jnp.maximum(orv, 0.0)
                z1[...] = z1v
                z1bf[...] = z1v.astype(jnp.bfloat16)

        @pl.when(p == 1)
        def _():
            W1v = W1b[...].astype(jnp.bfloat16)
            W2v = W2b[...].astype(jnp.bfloat16)
            bias = b1b[...] + b2b[...]
            ylv = jax.lax.dot_general(
                a, z1bf[...], (((1,), (0,)), ((), ())),
                preferred_element_type=jnp.float32,
            )
            lm = ylv * y1[sl_i, :]
            ol2[sl_i, :] = (
                jnp.dot(ylv.astype(jnp.bfloat16), W1v, preferred_element_type=jnp.float32)
                + jnp.dot(lm.astype(jnp.bfloat16), W2v, preferred_element_type=jnp.float32)
                + bias
            )
            nl2[sl_i, :] = ylv + lm
            rdot_accum(y1bf[sl_i, :])

            @pl.when(i == gi - 1)
            def _():
                yrv = jnp.transpose(yrT[...])
                nr2 = yrv + yrv * z1[...]
                nr2bf[...] = nr2.astype(jnp.bfloat16)

        @pl.when(p == 2)
        def _():
            W1v = W1b[...].astype(jnp.bfloat16)
            W2v = W2b[...].astype(jnp.bfloat16)
            bias = b1b[...] + b2b[...]
            ylv = jax.lax.dot_general(
                a, nr2bf[...], (((1,), (0,)), ((), ())),
                preferred_element_type=jnp.float32,
            )
            lm = ylv * nl2[sl_i, :]
            y2v = (
                jnp.dot(ylv.astype(jnp.bfloat16), W1v, preferred_element_type=jnp.float32)
                + jnp.dot(lm.astype(jnp.bfloat16), W2v, preferred_element_type=jnp.float32)
                + bias + ol2[sl_i, :]
            )
            y2out[...] = jnp.maximum(y2v, 0.0)

    return body


def _mega(A16, ol0, or0, nl0, nr0, nlbf0, nrbf0,
          W1a, b1a, W2a, b2a, W1b, b1b, W2b, b2b):
    gi = N // BI
    a_spec = pl.BlockSpec((BI, N), lambda p, i: (i, 0))
    full = pl.BlockSpec((N, F), lambda p, i: (0, 0))
    wspec = pl.BlockSpec((F, F), lambda p, i: (0, 0))
    bspec = pl.BlockSpec((1, F), lambda p, i: (0, 0))
    in_specs = [a_spec] + [full] * 6 + [wspec, bspec, wspec, bspec] * 2
    scratch = [
        pltpu.VMEM((F, N), jnp.float32),
        pltpu.VMEM((N, F), jnp.float32),
        pltpu.VMEM((N, F), jnp.float32),
        pltpu.VMEM((N, F), jnp.float32),
        pltpu.VMEM((N, F), jnp.float32),
        pltpu.VMEM((N, F), jnp.bfloat16),
        pltpu.VMEM((N, F), jnp.bfloat16),
        pltpu.VMEM((N, F), jnp.bfloat16),
    ]
    return pl.pallas_call(
        _mega_body(gi, BI),
        grid=(3, gi),
        in_specs=in_specs,
        out_specs=pl.BlockSpec((BI, F), lambda p, i: (i, 0)),
        out_shape=jax.ShapeDtypeStruct((N, F), jnp.float32),
        scratch_shapes=scratch,
    )(A16, ol0, or0, nl0, nr0, nlbf0, nrbf0,
      W1a, b1a, W2a, b2a, W1b, b1b, W2b, b2b)


def kernel(l_feat, r_feat, network, W1a, b1a, W2a, b2a, W1b, b1b, W2b, b2b):
    lbf = l_feat.astype(jnp.bfloat16)
    rbf = r_feat.astype(jnp.bfloat16)
    b1a = b1a.reshape(1, F)
    b2a = b2a.reshape(1, F)
    b1b = b1b.reshape(1, F)
    b2b = b2b.reshape(1, F)

    A16, ol, orv, nl, nr, nlbf, nrbf = _hop(
        network, rbf, lbf, l_feat, r_feat, None, W1a, b1a, W2a, b2a,
        need_r=True, need_next=True, relu=False, bf_out=False, cast_a=True,
    )
    return _mega(
        A16, ol, orv, nl, nr, nlbf, nrbf,
        W1a, b1a, W2a, b2a, W1b, b1b, W2b, b2b,
    )


# speedup vs baseline: 2.3423x; 1.0208x over previous
import jax
import jax.numpy as jnp
from jax.experimental import pallas as pl
from jax.experimental.pallas import tpu as pltpu

N = 4096
F = 128
BI = 1024
BI_CAST = 512


def _make_hop_body(gi, bi, need_r, need_next, has_acc, relu, bf_out, cast_a):
    def body(*refs):
        it = iter(refs)
        A = next(it)
        x0b = next(it)
        x1s = next(it) if need_r else None
        x1f = next(it)
        x0f = next(it) if need_r else None
        accl = next(it) if has_acc else None
        accr = next(it) if (has_acc and need_r) else None
        W1 = next(it)
        b1 = next(it)
        W2 = next(it)
        b2 = next(it)
        a16out = next(it) if cast_a else None
        outl = next(it)
        outr = next(it) if need_r else None
        nl = next(it) if need_next else None
        nr = next(it) if need_next else None
        nlbf = next(it) if (need_next or bf_out) else None
        nrbf = next(it) if (need_next or bf_out) else None
        yr = next(it) if need_r else None

        i = pl.program_id(0)
        if cast_a:
            a = A[...].astype(jnp.bfloat16)
            a16out[...] = a
        else:
            a = A[...]

        W1v = W1[...].astype(jnp.bfloat16)
        W2v = W2[...].astype(jnp.bfloat16)
        bias = b1[...] + b2[...]

        ylv = jax.lax.dot_general(
            a, x0b[...], (((1,), (0,)), ((), ())),
            preferred_element_type=jnp.float32,
        )
        lm = ylv * x1f[...]
        ol = (
            jnp.dot(ylv.astype(jnp.bfloat16), W1v, preferred_element_type=jnp.float32)
            + jnp.dot(lm.astype(jnp.bfloat16), W2v, preferred_element_type=jnp.float32)
            + bias
        )
        if has_acc:
            ol = ol + accl[...]
        if relu:
            ol = jnp.maximum(ol, 0.0)
        outl[...] = ol
        if need_next:
            nx = ylv + lm
            nl[...] = nx
            nlbf[...] = nx.astype(jnp.bfloat16)
        if bf_out:
            nlbf[...] = ol.astype(jnp.bfloat16)

        if need_r:
            x1v = x1s[...]
            cj = N
            for jj in range(N // cj):
                sl = slice(jj * cj, (jj + 1) * cj)
                pT = jax.lax.dot_general(
                    x1v, a[:, sl], (((0,), (0,)), ((), ())),
                    preferred_element_type=jnp.float32,
                )

                @pl.when(i == 0)
                def _():
                    yr[:, sl] = pT

                @pl.when(i != 0)
                def _():
                    yr[:, sl] += pT

            @pl.when(i == gi - 1)
            def _():
                yrv = jnp.transpose(yr[...])
                rm = yrv * x0f[...]
                orv = (
                    jnp.dot(yrv.astype(jnp.bfloat16), W1v, preferred_element_type=jnp.float32)
                    + jnp.dot(rm.astype(jnp.bfloat16), W2v, preferred_element_type=jnp.float32)
                    + bias
                )
                if has_acc:
                    orv = orv + accr[...]
                if relu:
                    orv = jnp.maximum(orv, 0.0)
                outr[...] = orv
                if need_next:
                    nx = yrv + rm
                    nr[...] = nx
                    nrbf[...] = nx.astype(jnp.bfloat16)
                if bf_out:
                    nrbf[...] = orv.astype(jnp.bfloat16)

    return body


def _hop(A_in, x0bf, x1bf, x1f, x0f, accs, W1, b1, W2, b2,
         *, need_r, need_next, relu, bf_out, cast_a=False):
    has_acc = accs is not None
    bi = BI_CAST if cast_a else BI
    gi = N // bi
    full = pl.BlockSpec((N, F), lambda i: (0, 0))
    strip = pl.BlockSpec((bi, F), lambda i: (i, 0))
    wspec = pl.BlockSpec((F, F), lambda i: (0, 0))
    bspec = pl.BlockSpec((1, F), lambda i: (0, 0))
    in_specs = [
        pl.BlockSpec((bi, N), lambda i: (i, 0)),
        full,
    ]
    ops = [A_in, x0bf]
    if need_r:
        in_specs.append(strip)
        ops.append(x1bf)
    in_specs.append(strip)
    ops.append(x1f)
    if need_r:
        in_specs.append(full)
        ops.append(x0f)
    if has_acc:
        in_specs.append(strip)
        ops.append(accs[0])
        if need_r:
            in_specs.append(full)
            ops.append(accs[1])
    in_specs += [wspec, bspec, wspec, bspec]
    ops += [W1, b1, W2, b2]

    out_shape = []
    out_specs = []
    if cast_a:
        out_shape.append(jax.ShapeDtypeStruct((N, N), jnp.bfloat16))
        out_specs.append(pl.BlockSpec((bi, N), lambda i: (i, 0)))
    out_shape.append(jax.ShapeDtypeStruct((N, F), jnp.float32))
    out_specs.append(strip)
    if need_r:
        out_shape.append(jax.ShapeDtypeStruct((N, F), jnp.float32))
        out_specs.append(full)
    if need_next:
        out_shape += [jax.ShapeDtypeStruct((N, F), jnp.float32)] * 2
        out_specs += [strip, full]
    if need_next or bf_out:
        out_shape += [jax.ShapeDtypeStruct((N, F), jnp.bfloat16)] * 2
        out_specs += [strip, full]
    scratch = [pltpu.VMEM((F, N), jnp.float32)] if need_r else []

    return pl.pallas_call(
        _make_hop_body(gi, bi, need_r, need_next, has_acc, relu, bf_out, cast_a),
        grid=(gi,),
        in_specs=in_specs,
        out_specs=tuple(out_specs),
        out_shape=tuple(out_shape),
        scratch_shapes=scratch,
    )(*ops)


def _mega_body(gi, bi):
    def body(A16, ol0, or0, nl0, nr0, nlbf0, nrbf0,
             W1a, b1a, W2a, b2a, W1b, b1b, W2b, b2b,
             y2out,
             yrT, y1, z1, ol2, nl2, y1bf, z1bf, nr2bf):
        p = pl.program_id(0)
        i = pl.program_id(1)
        a = A16[...]
        sl_i = pl.ds(i * bi, bi)
        cj = N

        def rdot_accum(x1v):
            for jj in range(N // cj):
                sl = slice(jj * cj, (jj + 1) * cj)
                pT = jax.lax.dot_general(
                    x1v, a[:, sl], (((0,), (0,)), ((), ())),
                    preferred_element_type=jnp.float32,
                )

                @pl.when(i == 0)
                def _():
                    yrT[:, sl] = pT

                @pl.when(i != 0)
                def _():
                    yrT[:, sl] += pT

        @pl.when(p == 0)
        def _():
            W1v = W1a[...].astype(jnp.bfloat16)
            W2v = W2a[...].astype(jnp.bfloat16)
            bias = b1a[...] + b2a[...]
            ylv = jax.lax.dot_general(
                a, nrbf0[...], (((1,), (0,)), ((), ())),
                preferred_element_type=jnp.float32,
            )
            lm = ylv * nl0[sl_i, :]
            olv = (
                jnp.dot(ylv.astype(jnp.bfloat16), W1v, preferred_element_type=jnp.float32)
                + jnp.dot(lm.astype(jnp.bfloat16), W2v, preferred_element_type=jnp.float32)
                + bias + ol0[sl_i, :]
            )
            y1v = jnp.maximum(olv, 0.0)
            y1[sl_i, :] = y1v
            y1bf[sl_i, :] = y1v.astype(jnp.bfloat16)
            rdot_accum(nlbf0[sl_i, :])

            @pl.when(i == gi - 1)
            def _():
                yrv = jnp.transpose(yrT[...])
                rm = yrv * nr0[...]
                orv = (
                    jnp.dot(yrv.astype(jnp.bfloat16), W1v, preferred_element_type=jnp.float32)
                    + jnp.dot(rm.astype(jnp.bfloat16), W2v, preferred_element_type=jnp.float32)
                    + bias + or0[...]
                )
                z1v = jnp.maximum(orv, 0.0)
                z1[...] = z1v
                z1bf[...] = z1v.astype(jnp.bfloat16)

        @pl.when(p == 1)
        def _():
            W1v = W1b[...].astype(jnp.bfloat16)
            W2v = W2b[...].astype(jnp.bfloat16)
            bias = b1b[...] + b2b[...]
            ylv = jax.lax.dot_general(
                a, z1bf[...], (((1,), (0,)), ((), ())),
                preferred_element_type=jnp.float32,
            )
            lm = ylv * y1[sl_i, :]
            ol2[sl_i, :] = (
                jnp.dot(ylv.astype(jnp.bfloat16), W1v, preferred_element_type=jnp.float32)
                + jnp.dot(lm.astype(jnp.bfloat16), W2v, preferred_element_type=jnp.float32)
                + bias
            )
            nl2[sl_i, :] = ylv + lm
            rdot_accum(y1bf[sl_i, :])

            @pl.when(i == gi - 1)
            def _():
                yrv = jnp.transpose(yrT[...])
                nr2 = yrv + yrv * z1[...]
                nr2bf[...] = nr2.astype(jnp.bfloat16)

        @pl.when(p == 2)
        def _():
            W1v = W1b[...].astype(jnp.bfloat16)
            W2v = W2b[...].astype(jnp.bfloat16)
            bias = b1b[...] + b2b[...]
            ylv = jax.lax.dot_general(
                a, nr2bf[...], (((1,), (0,)), ((), ())),
                preferred_element_type=jnp.float32,
            )
            lm = ylv * nl2[sl_i, :]
            y2v = (
                jnp.dot(ylv.astype(jnp.bfloat16), W1v, preferred_element_type=jnp.float32)
                + jnp.dot(lm.astype(jnp.bfloat16), W2v, preferred_element_type=jnp.float32)
                + bias + ol2[sl_i, :]
            )
            y2out[...] = jnp.maximum(y2v, 0.0)

    return body


def _mega(A16, ol0, or0, nl0, nr0, nlbf0, nrbf0,
          W1a, b1a, W2a, b2a, W1b, b1b, W2b, b2b):
    gi = N // BI
    a_spec = pl.BlockSpec((BI, N), lambda p, i: (i, 0))
    full = pl.BlockSpec((N, F), lambda p, i: (0, 0))
    wspec = pl.BlockSpec((F, F), lambda p, i: (0, 0))
    bspec = pl.BlockSpec((1, F), lambda p, i: (0, 0))
    in_specs = [a_spec] + [full] * 6 + [wspec, bspec, wspec, bspec] * 2
    scratch = [
        pltpu.VMEM((F, N), jnp.float32),
        pltpu.VMEM((N, F), jnp.float32),
        pltpu.VMEM((N, F), jnp.float32),
        pltpu.VMEM((N, F), jnp.float32),
        pltpu.VMEM((N, F), jnp.float32),
        pltpu.VMEM((N, F), jnp.bfloat16),
        pltpu.VMEM((N, F), jnp.bfloat16),
        pltpu.VMEM((N, F), jnp.bfloat16),
    ]
    return pl.pallas_call(
        _mega_body(gi, BI),
        grid=(3, gi),
        in_specs=in_specs,
        out_specs=pl.BlockSpec((BI, F), lambda p, i: (i, 0)),
        out_shape=jax.ShapeDtypeStruct((N, F), jnp.float32),
        scratch_shapes=scratch,
    )(A16, ol0, or0, nl0, nr0, nlbf0, nrbf0,
      W1a, b1a, W2a, b2a, W1b, b1b, W2b, b2b)


def kernel(l_feat, r_feat, network, W1a, b1a, W2a, b2a, W1b, b1b, W2b, b2b):
    lbf = l_feat.astype(jnp.bfloat16)
    rbf = r_feat.astype(jnp.bfloat16)
    b1a = b1a.reshape(1, F)
    b2a = b2a.reshape(1, F)
    b1b = b1b.reshape(1, F)
    b2b = b2b.reshape(1, F)

    A16, ol, orv, nl, nr, nlbf, nrbf = _hop(
        network, rbf, lbf, l_feat, r_feat, None, W1a, b1a, W2a, b2a,
        need_r=True, need_next=True, relu=False, bf_out=False, cast_a=True,
    )
    return _mega(
        A16, ol, orv, nl, nr, nlbf, nrbf,
        W1a, b1a, W2a, b2a, W1b, b1b, W2b, b2b,
    )
